# Initial kernel scaffold; baseline (speedup 1.0000x reference)
import jax
import jax.numpy as jnp
from jax.experimental import pallas as pl


def kernel(x, edge_index, W1, b1, W2, b2, W3, b3, cb0, cb1, cb2):
    raise NotImplementedError("write your pallas kernel here")


# speedup vs baseline: 4.6496x; 4.6496x over previous
import functools

import jax
import jax.numpy as jnp
from jax import lax
from jax.experimental import pallas as pl
from jax.experimental.pallas import tpu as pltpu
from jax.experimental.pallas import tpu_sc as plsc

_N = 10000
_E = 160000
_NT = 16
_CH = 128
_NCH = 80
_EPT = _NCH * _CH
_EPAD = _NT * _EPT
_ECC = _EPAD // 2
_NCH2 = _NCH // 2
_EPT2 = _NCH2 * _CH
_ACC = 10240
_ZPT = _ACC // _NT
_WBA = 624
_WBL = _N - 15 * _WBA
_R = 1000
_CW = 0.25


def _sc_mesh():
    return plsc.VectorSubcoreMesh(core_axis_name="c", subcore_axis_name="s")


def _ds(start, size, mult):
    return pl.ds(pl.multiple_of(start, mult), size)


def _writeback(sid, acc_sh, out):
    @pl.when(sid < 15)
    def _():
        off = _ds(sid * _WBA, _WBA, 16)
        pltpu.sync_copy(acc_sh.at[off], out.at[off])

    @pl.when(sid == 15)
    def _():
        off = pl.ds(15 * _WBA, _WBL)
        pltpu.sync_copy(acc_sh.at[off], out.at[off])


def _make_deg_kernel():
    @functools.partial(
        pl.kernel,
        mesh=_sc_mesh(),
        out_type=jax.ShapeDtypeStruct((_N, 128), jnp.float32),
        scratch_types=[
            pltpu.VMEM((_CH,), jnp.int32),
            pltpu.VMEM((_CH, 128), jnp.float32),
            pltpu.VMEM_SHARED((_ACC, 128), jnp.float32),
        ],
    )
    def deg_kernel(dstp, ones_hbm, z_hbm, out, dst_v, ones_v, acc_sh):
        cid = lax.axis_index("c")
        sid = lax.axis_index("s")

        @pl.when(cid == 0)
        def _():
            pltpu.sync_copy(z_hbm, acc_sh.at[_ds(sid * _ZPT, _ZPT, 128)])
            pltpu.sync_copy(ones_hbm, ones_v)
            plsc.subcore_barrier()

            def chunk(g, carry):
                off = sid * _EPT + g * _CH
                pltpu.sync_copy(dstp.at[_ds(off, _CH, 128)], dst_v)
                pltpu.sync_copy(ones_v, acc_sh.at[dst_v], add=True)
                return carry

            lax.fori_loop(0, _NCH, chunk, 0)
            plsc.subcore_barrier()
            _writeback(sid, acc_sh, out)

    return deg_kernel


def _make_agg_fs_kernel():
    sds = jax.ShapeDtypeStruct((_N, 128), jnp.float32)

    @functools.partial(
        pl.kernel,
        mesh=_sc_mesh(),
        out_type=(sds, sds),
        scratch_types=[
            pltpu.VMEM((_CH,), jnp.int32),
            pltpu.VMEM((_CH,), jnp.int32),
            pltpu.VMEM((_CH, 128), jnp.float32),
            pltpu.VMEM_SHARED((_ACC, 128), jnp.float32),
            pltpu.SemaphoreType.DMA,
        ],
    )
    def agg_kernel(hs0, hs1, srcp, dstp, z_hbm, out0, out1,
                   src_v, dst_v, rows_v, acc_sh, sem):
        cid = lax.axis_index("c")
        sid = lax.axis_index("s")

        def run(hs, out):
            pltpu.sync_copy(z_hbm, acc_sh.at[_ds(sid * _ZPT, _ZPT, 128)])
            plsc.subcore_barrier()

            def chunk(g, carry):
                off = sid * _EPT + g * _CH
                pltpu.sync_copy(srcp.at[_ds(off, _CH, 128)], src_v)
                pltpu.sync_copy(dstp.at[_ds(off, _CH, 128)], dst_v)
                pltpu.async_copy(hs.at[src_v], rows_v, sem).wait()
                pltpu.sync_copy(rows_v, acc_sh.at[dst_v], add=True)
                return carry

            lax.fori_loop(0, _NCH, chunk, 0)
            plsc.subcore_barrier()
            _writeback(sid, acc_sh, out)

        @pl.when(cid == 0)
        def _():
            run(hs0, out0)

        @pl.when(cid == 1)
        def _():
            run(hs1, out1)

    return agg_kernel


def _make_agg_es_kernel():
    sds = jax.ShapeDtypeStruct((_N, 128), jnp.float32)

    @functools.partial(
        pl.kernel,
        mesh=_sc_mesh(),
        out_type=(sds, sds),
        scratch_types=[
            pltpu.VMEM((_CH,), jnp.int32),
            pltpu.VMEM((_CH,), jnp.int32),
            pltpu.VMEM((_CH, 128), jnp.float32),
            pltpu.VMEM_SHARED((_ACC, 128), jnp.float32),
            pltpu.SemaphoreType.DMA,
        ],
    )
    def agg_kernel(hs, srcp, dstp, z_hbm, out0, out1,
                   src_v, dst_v, rows_v, acc_sh, sem):
        cid = lax.axis_index("c")
        sid = lax.axis_index("s")

        pltpu.sync_copy(z_hbm, acc_sh.at[_ds(sid * _ZPT, _ZPT, 128)])
        plsc.subcore_barrier()

        def chunk(g, carry):
            off = cid * _ECC + sid * _EPT2 + g * _CH
            pltpu.sync_copy(srcp.at[_ds(off, _CH, 128)], src_v)
            pltpu.sync_copy(dstp.at[_ds(off, _CH, 128)], dst_v)
            pltpu.async_copy(hs.at[src_v], rows_v, sem).wait()
            pltpu.sync_copy(rows_v, acc_sh.at[dst_v], add=True)
            return carry

        lax.fori_loop(0, _NCH2, chunk, 0)
        plsc.subcore_barrier()

        @pl.when(cid == 0)
        def _():
            _writeback(sid, acc_sh, out0)

        @pl.when(cid == 1)
        def _():
            _writeback(sid, acc_sh, out1)

    return agg_kernel


def _selu(x):
    alpha = 1.6732632423543772848170429916717
    scale = 1.0507009873554804934193349852946
    return scale * jnp.where(x > 0, x, alpha * (jnp.exp(x) - 1.0))


def _rvq(h, cb):
    residual = h
    sse = jnp.float32(0.0)
    idxs = []
    for i in range(cb.shape[0]):
        c = cb[i]
        cn = c / (jnp.sqrt(jnp.sum(c * c, axis=1, keepdims=True)) + 1e-8)
        rn = residual / (jnp.sqrt(jnp.sum(residual * residual, axis=1,
                                          keepdims=True)) + 1e-8)
        sim = lax.dot_general(rn, cn, (((1,), (1,)), ((), ())),
                              preferred_element_type=jnp.float32)
        m = jnp.max(sim, axis=1, keepdims=True)
        io = lax.broadcasted_iota(jnp.int32, sim.shape, 1)
        idx = jnp.min(jnp.where(sim >= m, io, jnp.int32(1 << 30)), axis=1)
        oh = (io == idx[:, None]).astype(jnp.float32)
        q = lax.dot_general(oh, cn, (((1,), (0,)), ((), ())),
                            precision=lax.Precision.HIGHEST,
                            preferred_element_type=jnp.float32)
        diff = q - residual
        sse = sse + jnp.sum(diff * diff)
        residual = residual - q
        idxs.append(idx)
    return idxs, sse


def _mm_body(x_ref, w_ref, o_ref):
    o_ref[...] = jnp.dot(x_ref[...], w_ref[...],
                         preferred_element_type=jnp.float32)


def _scale_body(deg_ref, h_ref, dinv_ref, o0_ref, o1_ref):
    deg = deg_ref[...][:, 0:1] + 1.0
    dinv = lax.rsqrt(deg)
    dinv_ref[...] = dinv
    h = h_ref[...] * dinv
    half = h.shape[1] // 2
    o0_ref[...] = h[:, :half]
    o1_ref[...] = h[:, half:]


def _stage_b_body(dinv_ref, a0, a1, s0, s1, b_ref, cb_ref, w_ref,
                  i0, i1, i2, sse_ref, o_ref):
    k = pl.program_id(0)
    dinv = dinv_ref[...]
    h = jnp.concatenate([a0[...] + s0[...], a1[...] + s1[...]], axis=1)
    h = _selu(h * dinv + b_ref[...])
    idxs, sse = _rvq(h, cb_ref[...])
    i0[...] = idxs[0][:, None]
    i1[...] = idxs[1][:, None]
    i2[...] = idxs[2][:, None]

    @pl.when(k == 0)
    def _():
        sse_ref[0, 0] = 0.0

    sse_ref[0, 0] += sse
    o_ref[...] = jnp.dot(h, w_ref[...],
                         preferred_element_type=jnp.float32) * dinv


def _stage_c_body(dinv_ref, p0, p1, s_ref, b_ref, cb_ref, w_ref,
                  i0, i1, i2, sse_ref, o_ref):
    k = pl.program_id(0)
    dinv = dinv_ref[...]
    h = p0[...] + p1[...] + s_ref[...]
    h = _selu(h * dinv + b_ref[...])
    idxs, sse = _rvq(h, cb_ref[...])
    i0[...] = idxs[0][:, None]
    i1[...] = idxs[1][:, None]
    i2[...] = idxs[2][:, None]

    @pl.when(k == 0)
    def _():
        sse_ref[0, 0] = 0.0

    sse_ref[0, 0] += sse
    o_ref[...] = jnp.dot(h, w_ref[...],
                         preferred_element_type=jnp.float32) * dinv


def _final_stage_body(dinv_ref, p0, p1, s_ref, b_ref, cb_ref,
                      i0, i1, i2, sse_ref, h_ref, ssum_ref):
    k = pl.program_id(0)
    h = p0[...] + p1[...] + s_ref[...]
    h = h * dinv_ref[...] + b_ref[...]
    idxs, sse = _rvq(h, cb_ref[...])
    i0[...] = idxs[0][:, None]
    i1[...] = idxs[1][:, None]
    i2[...] = idxs[2][:, None]

    @pl.when(k == 0)
    def _():
        sse_ref[0, 0] = 0.0
        ssum_ref[0, 0] = 0.0

    sse_ref[0, 0] += sse
    ssum_ref[0, 0] += jnp.sum(h)
    h_ref[...] = h


def _out_body(h_ref, s_ref, o_ref):
    t = jnp.tanh(h_ref[...] / s_ref[0, 0]) ** 2
    dn = jnp.maximum(
        jnp.sqrt(jnp.sum(t * t, axis=1, keepdims=True)), 1e-12)
    o_ref[...] = t / dn


def _row_spec(cols):
    return pl.BlockSpec((_R, cols), lambda i: (i, 0))


def _full_spec(shape):
    ndim = len(shape)
    return pl.BlockSpec(shape, lambda i: (0,) * ndim)


def _smem_spec():
    return pl.BlockSpec(memory_space=pltpu.SMEM)


_GRID = _N // _R
_F32 = jnp.float32
_I32 = jnp.int32


def _mm1(x, W1):
    return pl.pallas_call(
        _mm_body,
        grid=(_GRID,),
        in_specs=[_row_spec(256), _full_spec((256, 256))],
        out_specs=_row_spec(256),
        out_shape=jax.ShapeDtypeStruct((_N, 256), _F32),
    )(x, W1)


def _scale_split(deg16, h1p):
    return pl.pallas_call(
        _scale_body,
        grid=(_GRID,),
        in_specs=[_row_spec(128), _row_spec(256)],
        out_specs=[_row_spec(1), _row_spec(128), _row_spec(128)],
        out_shape=[jax.ShapeDtypeStruct((_N, 1), _F32),
                   jax.ShapeDtypeStruct((_N, 128), _F32),
                   jax.ShapeDtypeStruct((_N, 128), _F32)],
    )(deg16, h1p)


def _stage_b(dinv, a0, a1, s0, s1, b, cb, W):
    return pl.pallas_call(
        _stage_b_body,
        grid=(_GRID,),
        in_specs=[_row_spec(1), _row_spec(128), _row_spec(128),
                  _row_spec(128), _row_spec(128),
                  _full_spec((1, 256)), _full_spec((3, 16, 256)),
                  _full_spec((256, 128))],
        out_specs=[_row_spec(1), _row_spec(1), _row_spec(1),
                   _smem_spec(), _row_spec(128)],
        out_shape=[jax.ShapeDtypeStruct((_N, 1), _I32),
                   jax.ShapeDtypeStruct((_N, 1), _I32),
                   jax.ShapeDtypeStruct((_N, 1), _I32),
                   jax.ShapeDtypeStruct((1, 1), _F32),
                   jax.ShapeDtypeStruct((_N, 128), _F32)],
    )(dinv, a0, a1, s0, s1, b.reshape(1, 256), cb, W)


def _stage_c(dinv, p0, p1, s, b, cb, W):
    return pl.pallas_call(
        _stage_c_body,
        grid=(_GRID,),
        in_specs=[_row_spec(1), _row_spec(128), _row_spec(128),
                  _row_spec(128),
                  _full_spec((1, 128)), _full_spec((3, 16, 128)),
                  _full_spec((128, 128))],
        out_specs=[_row_spec(1), _row_spec(1), _row_spec(1),
                   _smem_spec(), _row_spec(128)],
        out_shape=[jax.ShapeDtypeStruct((_N, 1), _I32),
                   jax.ShapeDtypeStruct((_N, 1), _I32),
                   jax.ShapeDtypeStruct((_N, 1), _I32),
                   jax.ShapeDtypeStruct((1, 1), _F32),
                   jax.ShapeDtypeStruct((_N, 128), _F32)],
    )(dinv, p0, p1, s, b.reshape(1, 128), cb, W)


def _final_stage(dinv, p0, p1, s, b, cb):
    return pl.pallas_call(
        _final_stage_body,
        grid=(_GRID,),
        in_specs=[_row_spec(1), _row_spec(128), _row_spec(128),
                  _row_spec(128),
                  _full_spec((1, 128)), _full_spec((3, 16, 128))],
        out_specs=[_row_spec(1), _row_spec(1), _row_spec(1),
                   _smem_spec(), _row_spec(128), _smem_spec()],
        out_shape=[jax.ShapeDtypeStruct((_N, 1), _I32),
                   jax.ShapeDtypeStruct((_N, 1), _I32),
                   jax.ShapeDtypeStruct((_N, 1), _I32),
                   jax.ShapeDtypeStruct((1, 1), _F32),
                   jax.ShapeDtypeStruct((_N, 128), _F32),
                   jax.ShapeDtypeStruct((1, 1), _F32)],
    )(dinv, p0, p1, s, b.reshape(1, 128), cb)


def _final_out(h3, ssum):
    return pl.pallas_call(
        _out_body,
        grid=(_GRID,),
        in_specs=[_row_spec(128), _smem_spec()],
        out_specs=_row_spec(128),
        out_shape=jax.ShapeDtypeStruct((_N, 128), _F32),
    )(h3, ssum)


def kernel(x, edge_index, W1, b1, W2, b2, W3, b3, cb0, cb1, cb2):
    src = edge_index[0].astype(_I32)
    dst = edge_index[1].astype(_I32)
    pad = _EPAD - _E
    srcp = jnp.concatenate([src, jnp.zeros((pad,), _I32)])
    dstp = jnp.concatenate([dst, jnp.full((pad,), _N, _I32)])

    deg_k = _make_deg_kernel()
    agg_fs = _make_agg_fs_kernel()
    agg_es = _make_agg_es_kernel()

    ones128 = jnp.ones((_CH, 128), _F32)
    z128 = jnp.zeros((_ZPT, 128), _F32)

    deg16 = deg_k(dstp, ones128, z128)
    h1p = _mm1(x, W1)
    dinv, hs1_0, hs1_1 = _scale_split(deg16, h1p)

    agg1_0, agg1_1 = agg_fs(hs1_0, hs1_1, srcp, dstp, z128)
    ia0, ia1, ia2, sse0, hs2 = _stage_b(
        dinv, agg1_0, agg1_1, hs1_0, hs1_1, b1, cb0, W2)

    agg2_0, agg2_1 = agg_es(hs2, srcp, dstp, z128)
    ib0, ib1, ib2, sse1, hs3 = _stage_c(
        dinv, agg2_0, agg2_1, hs2, b2, cb1, W3)

    agg3_0, agg3_1 = agg_es(hs3, srcp, dstp, z128)
    ic0, ic1, ic2, sse2, h3, ssum = _final_stage(
        dinv, agg3_0, agg3_1, hs3, b3, cb2)

    out = _final_out(h3, ssum)

    total_loss = _CW * (sse0[0, 0] / (_N * 256)
                        + sse1[0, 0] / (_N * 128)
                        + sse2[0, 0] / (_N * 128))
    ids = jnp.concatenate(
        [ia0, ia1, ia2, ib0, ib1, ib2, ic0, ic1, ic2], axis=1)
    return (out, total_loss, ids)


# speedup vs baseline: 6.4560x; 1.3885x over previous
import functools

import jax
import jax.numpy as jnp
from jax import lax
from jax.experimental import pallas as pl
from jax.experimental.pallas import tpu as pltpu
from jax.experimental.pallas import tpu_sc as plsc

_N = 10000
_E = 160000
_NT = 16
_CH = 128
_NCH = 84
_EPT = _NCH * _CH
_EPAD = _NT * _EPT
_ACC = 10112
_ZPT = _ACC // _NT
_WBA = 624
_WBL = _N - 15 * _WBA
_R = 1000
_CW = 0.25


def _sc_mesh():
    return plsc.VectorSubcoreMesh(core_axis_name="c", subcore_axis_name="s")


def _ds(start, size, mult):
    return pl.ds(pl.multiple_of(start, mult), size)


def _writeback(sid, acc_sh, out):
    @pl.when(sid < 15)
    def _():
        off = _ds(sid * _WBA, _WBA, 16)
        pltpu.sync_copy(acc_sh.at[off], out.at[off])

    @pl.when(sid == 15)
    def _():
        off = pl.ds(15 * _WBA, _WBL)
        pltpu.sync_copy(acc_sh.at[off], out.at[off])


def _make_deg_kernel():
    @functools.partial(
        pl.kernel,
        mesh=_sc_mesh(),
        out_type=jax.ShapeDtypeStruct((_N, 128), jnp.float32),
        scratch_types=[
            pltpu.VMEM((_CH,), jnp.int32),
            pltpu.VMEM((_CH,), jnp.int32),
            pltpu.VMEM((_CH,), jnp.int32),
            pltpu.VMEM((_CH,), jnp.int32),
            pltpu.VMEM((_CH, 128), jnp.float32),
            pltpu.VMEM_SHARED((_ACC, 128), jnp.float32),
            pltpu.SemaphoreType.DMA,
            pltpu.SemaphoreType.DMA,
            pltpu.SemaphoreType.DMA,
            pltpu.SemaphoreType.DMA,
            pltpu.SemaphoreType.DMA,
            pltpu.SemaphoreType.DMA,
        ],
    )
    def deg_kernel(dstp, ones_hbm, z_hbm, out,
                   d0, d1, d2, d3, ones_v, acc_sh,
                   si0, si1, si2, si3, ss0, ss1):
        cid = lax.axis_index("c")
        sid = lax.axis_index("s")
        dv = (d0, d1, d2, d3)
        semi = (si0, si1, si2, si3)
        sems = (ss0, ss1)

        def idx_load(g, k):
            pltpu.async_copy(dstp.at[_ds(sid * _EPT + g * _CH, _CH, 128)],
                             dv[k], semi[k])

        def idx_wait(g, k):
            pltpu.make_async_copy(
                dstp.at[_ds(sid * _EPT + g * _CH, _CH, 128)],
                dv[k], semi[k]).wait()

        @pl.when(cid == 0)
        def _():
            pltpu.sync_copy(z_hbm, acc_sh.at[_ds(sid * _ZPT, _ZPT, 8)])
            pltpu.sync_copy(ones_hbm, ones_v)
            plsc.subcore_barrier()
            idx_load(0, 0)
            idx_load(1, 1)

            def quad(q, carry):
                for b in range(4):
                    g = q * 4 + b

                    @pl.when(g >= 2)
                    def _():
                        pltpu.make_async_copy(
                            ones_v, acc_sh.at[dv[(b + 2) % 4]],
                            sems[b % 2]).wait()

                    idx_wait(g, b)
                    pltpu.async_copy(ones_v, acc_sh.at[dv[b]],
                                     sems[b % 2], add=True)

                    @pl.when(g + 2 < _NCH)
                    def _():
                        idx_load(g + 2, (b + 2) % 4)
                return carry

            lax.fori_loop(0, _NCH // 4, quad, 0)
            for b in (2, 3):
                pltpu.make_async_copy(ones_v, acc_sh.at[dv[b]],
                                      sems[b % 2]).wait()
            plsc.subcore_barrier()
            _writeback(sid, acc_sh, out)

    return deg_kernel


def _agg_pipeline(nch, base, srcp, dstp, hs, sv, dv, rows, semi, semg,
                  sems, acc_sh):
    def idx_load(g, k):
        off = _ds(base + g * _CH, _CH, 128)
        pltpu.async_copy(srcp.at[off], sv[k], semi[k])
        pltpu.async_copy(dstp.at[off], dv[k], semi[k])

    def idx_wait(g, k):
        off = _ds(base + g * _CH, _CH, 128)
        pltpu.make_async_copy(srcp.at[off], sv[k], semi[k]).wait()
        pltpu.make_async_copy(dstp.at[off], dv[k], semi[k]).wait()

    idx_load(0, 0)
    idx_load(1, 1)
    idx_wait(0, 0)
    pltpu.async_copy(hs.at[sv[0]], rows[0], semg[0])

    def quad(q, carry):
        for b in range(4):
            g = q * 4 + b
            pltpu.make_async_copy(hs.at[sv[b]], rows[b % 2],
                                  semg[b % 2]).wait()
            pltpu.async_copy(rows[b % 2], acc_sh.at[dv[b]],
                             sems[b % 2], add=True)

            @pl.when(g >= 1)
            def _():
                pltpu.make_async_copy(rows[(b + 1) % 2],
                                      acc_sh.at[dv[(b + 3) % 4]],
                                      sems[(b + 1) % 2]).wait()

            @pl.when(g + 1 < nch)
            def _():
                idx_wait(g + 1, (b + 1) % 4)
                pltpu.async_copy(hs.at[sv[(b + 1) % 4]],
                                 rows[(b + 1) % 2], semg[(b + 1) % 2])

            @pl.when(g + 2 < nch)
            def _():
                idx_load(g + 2, (b + 2) % 4)
        return carry

    lax.fori_loop(0, nch // 4, quad, 0)
    pltpu.make_async_copy(rows[1], acc_sh.at[dv[3]], sems[1]).wait()


_AGG_SCRATCH = [
    pltpu.VMEM((_CH,), jnp.int32),
    pltpu.VMEM((_CH,), jnp.int32),
    pltpu.VMEM((_CH,), jnp.int32),
    pltpu.VMEM((_CH,), jnp.int32),
    pltpu.VMEM((_CH,), jnp.int32),
    pltpu.VMEM((_CH,), jnp.int32),
    pltpu.VMEM((_CH,), jnp.int32),
    pltpu.VMEM((_CH,), jnp.int32),
    pltpu.VMEM((_CH, 128), jnp.float32),
    pltpu.VMEM((_CH, 128), jnp.float32),
    pltpu.VMEM_SHARED((_ACC, 128), jnp.float32),
] + [pltpu.SemaphoreType.DMA] * 8


def _make_agg_fs_kernel():
    sds = jax.ShapeDtypeStruct((_N, 128), jnp.float32)

    @functools.partial(
        pl.kernel,
        mesh=_sc_mesh(),
        out_type=(sds, sds),
        scratch_types=list(_AGG_SCRATCH),
    )
    def agg_kernel(hs0, hs1, srcp, dstp, z_hbm, out0, out1,
                   s0, s1, s2, s3, d0, d1, d2, d3, rows0, rows1, acc_sh,
                   i0, i1, i2, i3, g0, g1, c0, c1):
        cid = lax.axis_index("c")
        sid = lax.axis_index("s")

        def run(hs, out):
            pltpu.sync_copy(z_hbm, acc_sh.at[_ds(sid * _ZPT, _ZPT, 8)])
            plsc.subcore_barrier()
            _agg_pipeline(_NCH, sid * _EPT, srcp, dstp, hs,
                          (s0, s1, s2, s3), (d0, d1, d2, d3),
                          (rows0, rows1), (i0, i1, i2, i3), (g0, g1),
                          (c0, c1), acc_sh)
            plsc.subcore_barrier()
            _writeback(sid, acc_sh, out)

        @pl.when(cid == 0)
        def _():
            run(hs0, out0)

        @pl.when(cid == 1)
        def _():
            run(hs1, out1)

    return agg_kernel


def _make_agg_es_kernel():
    @functools.partial(
        pl.kernel,
        mesh=_sc_mesh(),
        out_type=jax.ShapeDtypeStruct((_N, 128), jnp.float32),
        scratch_types=list(_AGG_SCRATCH),
    )
    def agg_kernel(hs, srcp, dstp, z_hbm, out,
                   s0, s1, s2, s3, d0, d1, d2, d3, rows0, rows1, acc_sh,
                   i0, i1, i2, i3, g0, g1, c0, c1):
        cid = lax.axis_index("c")
        sid = lax.axis_index("s")

        @pl.when(cid == 0)
        def _():
            pltpu.sync_copy(z_hbm, acc_sh.at[_ds(sid * _ZPT, _ZPT, 8)])
            plsc.subcore_barrier()
            _agg_pipeline(_NCH, sid * _EPT, srcp, dstp, hs,
                          (s0, s1, s2, s3), (d0, d1, d2, d3),
                          (rows0, rows1), (i0, i1, i2, i3), (g0, g1),
                          (c0, c1), acc_sh)
            plsc.subcore_barrier()
            _writeback(sid, acc_sh, out)

    return agg_kernel


def _selu(x):
    alpha = 1.6732632423543772848170429916717
    scale = 1.0507009873554804934193349852946
    return scale * jnp.where(x > 0, x, alpha * (jnp.exp(x) - 1.0))


def _rvq(h, cb):
    residual = h
    sse = jnp.float32(0.0)
    idxs = []
    for i in range(cb.shape[0]):
        cn = cb[i]
        rn = residual / (jnp.sqrt(jnp.sum(residual * residual, axis=1,
                                          keepdims=True)) + 1e-8)
        sim = lax.dot_general(rn, cn, (((1,), (1,)), ((), ())),
                              preferred_element_type=jnp.float32)
        m = jnp.max(sim, axis=1, keepdims=True)
        io = lax.broadcasted_iota(jnp.int32, sim.shape, 1)
        idx = jnp.min(jnp.where(sim >= m, io, jnp.int32(1 << 30)), axis=1)
        oh = (io == idx[:, None]).astype(jnp.float32)
        q = lax.dot_general(oh, cn, (((1,), (0,)), ((), ())),
                            precision=lax.Precision.HIGHEST,
                            preferred_element_type=jnp.float32)
        diff = q - residual
        sse = sse + jnp.sum(diff * diff)
        residual = residual - q
        idxs.append(idx)
    return idxs, sse


def _mm_body(x_ref, w_ref, o_ref):
    o_ref[...] = jnp.dot(x_ref[...], w_ref[...],
                         preferred_element_type=jnp.float32)


def _scale_body(dinv_ref, h_ref, o0_ref, o1_ref):
    h = h_ref[...] * dinv_ref[...]
    half = h.shape[1] // 2
    o0_ref[...] = h[:, :half]
    o1_ref[...] = h[:, half:]


def _pre_b_body(dinv_ref, a0, a1, b_ref, o_ref):
    h = jnp.concatenate([a0[...], a1[...]], axis=1)
    o_ref[...] = h * dinv_ref[...] + b_ref[...]


def _pre_c_body(dinv_ref, p_ref, b_ref, o_ref):
    o_ref[...] = p_ref[...] * dinv_ref[...] + b_ref[...]


def _post_body(dinv_ref, h_ref, cb_ref, w_ref, i0, i1, i2, sse_ref, o_ref):
    k = pl.program_id(0)
    h = h_ref[...]
    idxs, sse = _rvq(h, cb_ref[...])
    i0[...] = idxs[0][:, None]
    i1[...] = idxs[1][:, None]
    i2[...] = idxs[2][:, None]

    @pl.when(k == 0)
    def _():
        sse_ref[0, 0] = 0.0

    sse_ref[0, 0] += sse
    o_ref[...] = jnp.dot(h, w_ref[...],
                         preferred_element_type=jnp.float32) * dinv_ref[...]


def _final_stage_body(dinv_ref, p_ref, b_ref, cb_ref,
                      i0, i1, i2, sse_ref, h_ref, ssum_ref):
    k = pl.program_id(0)
    h = p_ref[...] * dinv_ref[...] + b_ref[...]
    idxs, sse = _rvq(h, cb_ref[...])
    i0[...] = idxs[0][:, None]
    i1[...] = idxs[1][:, None]
    i2[...] = idxs[2][:, None]

    @pl.when(k == 0)
    def _():
        sse_ref[0, 0] = 0.0
        ssum_ref[0, 0] = 0.0

    sse_ref[0, 0] += sse
    ssum_ref[0, 0] += jnp.sum(h)
    h_ref[...] = h


def _out_body(h_ref, s_ref, o_ref):
    t = jnp.tanh(h_ref[...] / s_ref[0, 0]) ** 2
    dn = jnp.maximum(
        jnp.sqrt(jnp.sum(t * t, axis=1, keepdims=True)), 1e-12)
    o_ref[...] = t / dn


def _row_spec(cols):
    return pl.BlockSpec((_R, cols), lambda i: (i, 0))


def _full_spec(shape):
    ndim = len(shape)
    return pl.BlockSpec(shape, lambda i: (0,) * ndim)


def _smem_spec():
    return pl.BlockSpec(memory_space=pltpu.SMEM)


_GRID = _N // _R
_F32 = jnp.float32
_I32 = jnp.int32


def _mm1(x, W1):
    return pl.pallas_call(
        _mm_body,
        grid=(_GRID,),
        in_specs=[_row_spec(256), _full_spec((256, 256))],
        out_specs=_row_spec(256),
        out_shape=jax.ShapeDtypeStruct((_N, 256), _F32),
    )(x, W1)


def _scale_split(dinv, h1p):
    return pl.pallas_call(
        _scale_body,
        grid=(_GRID,),
        in_specs=[_row_spec(1), _row_spec(256)],
        out_specs=[_row_spec(128), _row_spec(128)],
        out_shape=[jax.ShapeDtypeStruct((_N, 128), _F32),
                   jax.ShapeDtypeStruct((_N, 128), _F32)],
    )(dinv, h1p)


def _pre_b(dinv, a0, a1, b):
    return pl.pallas_call(
        _pre_b_body,
        grid=(_GRID,),
        in_specs=[_row_spec(1), _row_spec(128), _row_spec(128),
                  _full_spec((1, 256))],
        out_specs=_row_spec(256),
        out_shape=jax.ShapeDtypeStruct((_N, 256), _F32),
    )(dinv, a0, a1, b.reshape(1, 256))


def _pre_c(dinv, p, b):
    return pl.pallas_call(
        _pre_c_body,
        grid=(_GRID,),
        in_specs=[_row_spec(1), _row_spec(128), _full_spec((1, 128))],
        out_specs=_row_spec(128),
        out_shape=jax.ShapeDtypeStruct((_N, 128), _F32),
    )(dinv, p, b.reshape(1, 128))


def _post(dinv, h, cb, W):
    d = h.shape[1]
    dn = W.shape[1]
    return pl.pallas_call(
        _post_body,
        grid=(_GRID,),
        in_specs=[_row_spec(1), _row_spec(d), _full_spec((3, 16, d)),
                  _full_spec((d, dn))],
        out_specs=[_row_spec(1), _row_spec(1), _row_spec(1),
                   _smem_spec(), _row_spec(dn)],
        out_shape=[jax.ShapeDtypeStruct((_N, 1), _I32),
                   jax.ShapeDtypeStruct((_N, 1), _I32),
                   jax.ShapeDtypeStruct((_N, 1), _I32),
                   jax.ShapeDtypeStruct((1, 1), _F32),
                   jax.ShapeDtypeStruct((_N, dn), _F32)],
    )(dinv, h, cb, W)


def _final_stage(dinv, p, b, cb):
    return pl.pallas_call(
        _final_stage_body,
        grid=(_GRID,),
        in_specs=[_row_spec(1), _row_spec(128),
                  _full_spec((1, 128)), _full_spec((3, 16, 128))],
        out_specs=[_row_spec(1), _row_spec(1), _row_spec(1),
                   _smem_spec(), _row_spec(128), _smem_spec()],
        out_shape=[jax.ShapeDtypeStruct((_N, 1), _I32),
                   jax.ShapeDtypeStruct((_N, 1), _I32),
                   jax.ShapeDtypeStruct((_N, 1), _I32),
                   jax.ShapeDtypeStruct((1, 1), _F32),
                   jax.ShapeDtypeStruct((_N, 128), _F32),
                   jax.ShapeDtypeStruct((1, 1), _F32)],
    )(dinv, p, b.reshape(1, 128), cb)


def _final_out(h3, ssum):
    return pl.pallas_call(
        _out_body,
        grid=(_GRID,),
        in_specs=[_row_spec(128), _smem_spec()],
        out_specs=_row_spec(128),
        out_shape=jax.ShapeDtypeStruct((_N, 128), _F32),
    )(h3, ssum)


def kernel(x, edge_index, W1, b1, W2, b2, W3, b3, cb0, cb1, cb2):
    src = edge_index[0].astype(_I32)
    dst = edge_index[1].astype(_I32)
    loop = jnp.arange(_N, dtype=_I32)
    pad = _EPAD - _E - _N
    srcp = jnp.concatenate([src, loop, jnp.zeros((pad,), _I32)])
    dstp = jnp.concatenate([dst, loop, jnp.full((pad,), _N, _I32)])

    def _cn(cb):
        return cb / (jnp.linalg.norm(cb, axis=-1, keepdims=True) + 1e-8)

    cn0, cn1, cn2 = _cn(cb0), _cn(cb1), _cn(cb2)

    deg_k = _make_deg_kernel()
    agg_fs = _make_agg_fs_kernel()
    agg_es = _make_agg_es_kernel()

    ones128 = jnp.ones((_CH, 128), _F32)
    z128 = jnp.zeros((_ZPT, 128), _F32)

    deg16 = deg_k(dstp, ones128, z128)
    deg = deg16[:, 0]
    dinv = jnp.where(deg > 0, lax.rsqrt(jnp.maximum(deg, 1e-12)),
                     0.0).reshape(_N, 1)
    h1p = _mm1(x, W1)
    hs1_0, hs1_1 = _scale_split(dinv, h1p)

    agg1_0, agg1_1 = agg_fs(hs1_0, hs1_1, srcp, dstp, z128)
    h1 = jax.nn.selu(_pre_b(dinv, agg1_0, agg1_1, b1))
    ia0, ia1, ia2, sse0, hs2 = _post(dinv, h1, cn0, W2)

    agg2 = agg_es(hs2, srcp, dstp, z128)
    h2 = jax.nn.selu(_pre_c(dinv, agg2, b2))
    ib0, ib1, ib2, sse1, hs3 = _post(dinv, h2, cn1, W3)

    agg3 = agg_es(hs3, srcp, dstp, z128)
    ic0, ic1, ic2, sse2, h3, ssum = _final_stage(dinv, agg3, b3, cn2)

    out = _final_out(h3, ssum)

    total_loss = _CW * (sse0[0, 0] / (_N * 256)
                        + sse1[0, 0] / (_N * 128)
                        + sse2[0, 0] / (_N * 128))
    ids = jnp.concatenate(
        [ia0, ia1, ia2, ib0, ib1, ib2, ic0, ic1, ic2], axis=1)
    return (out, total_loss, ids)
